# Initial kernel scaffold; baseline (speedup 1.0000x reference)
import jax
import jax.numpy as jnp
from jax.experimental import pallas as pl


def kernel(q, k, v):
    raise NotImplementedError("write your pallas kernel here")


# speedup vs baseline: 4.0972x; 4.0972x over previous
import jax
import jax.numpy as jnp
from jax import lax
from jax.experimental import pallas as pl

_W = 64


def _windowed_sort_kernel(v_ref, o_ref):
    x = v_ref[0]
    L, C = x.shape
    ti = lax.broadcasted_iota(jnp.int32, (L, C), 0)
    ci = lax.broadcasted_iota(jnp.int32, (L, C), 1)
    r = (ti - ci) & (_W - 1)

    k = 2
    while k <= _W:
        j = k // 2
        while j > 0:
            up = jnp.concatenate([x[j:], x[:j]], axis=0)
            dn = jnp.concatenate([x[-j:], x[:-j]], axis=0)
            bitj = (r & j) == 0
            p = jnp.where(bitj, up, dn)
            take_min = ((r & k) == 0) == bitj
            x = jnp.where(take_min, jnp.minimum(x, p), jnp.maximum(x, p))
            j //= 2
        k *= 2
    o_ref[0] = x


def kernel(q, k, v):
    B, L, D = v.shape
    C = 256
    grid = (B, D // C)
    return pl.pallas_call(
        _windowed_sort_kernel,
        grid=grid,
        in_specs=[pl.BlockSpec((1, L, C), lambda b, c: (b, 0, c))],
        out_specs=pl.BlockSpec((1, L, C), lambda b, c: (b, 0, c)),
        out_shape=jax.ShapeDtypeStruct(v.shape, v.dtype),
    )(v)


# speedup vs baseline: 5.6800x; 1.3863x over previous
import jax
import jax.numpy as jnp
from jax import lax
from jax.experimental import pallas as pl
from jax.experimental.pallas import tpu as pltpu

_W = 64


def _windowed_sort_kernel(v_ref, o_ref):
    x = v_ref[0]
    L, C = x.shape
    ti = lax.broadcasted_iota(jnp.int32, (L, C), 0)
    ci = lax.broadcasted_iota(jnp.int32, (L, C), 1)
    r = (ti - ci) & (_W - 1)
    bit_zero = [(r & (1 << b)) == 0 for b in range(6)]

    k = 2
    while k <= _W:
        j = k // 2
        while j > 0:
            up = jnp.concatenate([x[j:], x[:j]], axis=0)
            dn = jnp.concatenate([x[-j:], x[:-j]], axis=0)
            bitj = bit_zero[j.bit_length() - 1]
            p = jnp.where(bitj, up, dn)
            if k == _W:
                take_min = bitj
            else:
                take_min = bit_zero[k.bit_length() - 1] == bitj
            x = jnp.where(take_min, jnp.minimum(x, p), jnp.maximum(x, p))
            j //= 2
        k *= 2
    o_ref[0] = x


def kernel(q, k, v):
    B, L, D = v.shape
    C = 256
    grid = (B, D // C)
    return pl.pallas_call(
        _windowed_sort_kernel,
        grid=grid,
        in_specs=[pl.BlockSpec((1, L, C), lambda b, c: (b, 0, c))],
        out_specs=pl.BlockSpec((1, L, C), lambda b, c: (b, 0, c)),
        out_shape=jax.ShapeDtypeStruct(v.shape, v.dtype),
        compiler_params=pltpu.CompilerParams(
            dimension_semantics=("parallel", "parallel"),
        ),
    )(v)
